# baseline (speedup 1.0000x reference)
import functools

import jax
import jax.numpy as jnp
from jax import lax
from jax.experimental import pallas as pl
from jax.experimental.pallas import tpu as pltpu
from jax.experimental.pallas import tpu_sc as plsc

BS = 64
NQ = 32768
K = NQ // 16
NBUCKET = 1024
NB_VREG = NBUCKET // 16


def _cent_body(x_ref, y_ref, box_ref, cent_ref):
    xb = x_ref[...]
    yb = y_ref[...]
    cx = box_ref[:, 0:1]
    cy = box_ref[:, 1:2]
    w = box_ref[:, 2:3]
    h = box_ref[:, 3:4]
    xmin = cx - w / 2.0
    ymin = cy - h / 2.0
    xmax = cx + w / 2.0
    ymax = cy + h / 2.0
    left = jnp.clip(xb - xmin, 0.0, 1.0)
    right = jnp.clip(xmax - xb, 0.0, 1.0)
    top = jnp.clip(yb - ymin, 0.0, 1.0)
    down = jnp.clip(ymax - yb, 0.0, 1.0)
    sx = (left + right) / 2.0
    dx = jnp.abs(left - right) / 2.0
    sy = (top + down) / 2.0
    dy = jnp.abs(top - down) / 2.0
    cxn = (sx - dx) / (sx + dx)
    cyn = (sy - dy) / (sy + dy)
    cent_ref[...] = jnp.sqrt(cxn * cyn)


def _mask_body(cent_ref, thr_ref, mask_ref):
    mask_ref[...] = cent_ref[...] > thr_ref[:, 0:1]


def _select_row(row_v, hists):
    lanes = lax.iota(jnp.int32, 16)
    ones = jnp.ones((16,), jnp.int32)
    zeros16 = jnp.zeros((16,), jnp.int32)
    nh = len(hists)

    prefix = jnp.int32(0)
    need = jnp.int32(K + 1)
    for p in range(3):
        shift = 20 - 10 * p

        def zero_body(j, _):
            for h_v in hists:
                h_v[pl.ds(j * 16, 16)] = zeros16
            return 0

        lax.fori_loop(0, NB_VREG, zero_body, 0, unroll=8)

        pfx = prefix

        def hist_body(i, _):
            for q in range(nh):
                idx = i * nh + q
                v = row_v[pl.ds(idx * 16, 16)]
                u = lax.bitcast_convert_type(v, jnp.int32)
                u = jnp.where(v == v, u, 0)
                bucket = lax.shift_right_logical(u, shift) & (NBUCKET - 1)
                m = u != 0
                if p != 0:
                    m = jnp.logical_and(
                        m, lax.shift_right_logical(u, shift + 10) == pfx)
                plsc.addupdate_scatter(hists[q], [bucket], ones, mask=m)
            return 0

        lax.fori_loop(0, NQ // (16 * nh), hist_body, 0, unroll=2)

        def scan_cond(carry):
            t, _, found, _, _ = carry
            return jnp.logical_and(t < NB_VREG, jnp.logical_not(found))

        def scan_body(carry):
            t, above, found, bsel, above_sel = carry
            j = NB_VREG - 1 - t
            h = hists[0][pl.ds(j * 16, 16)]
            for h_v in hists[1:]:
                h = h + h_v[pl.ds(j * 16, 16)]
            csum = plsc.cumsum(h)
            total = jnp.max(csum)
            s = total - csum + h
            ge = (above + s) >= need
            cnt = jnp.max(plsc.all_reduce_population_count(ge))
            istar = cnt - 1
            h_at = jnp.max(jnp.where(lanes == istar, h, 0))
            s_at = jnp.max(jnp.where(lanes == istar, s, jnp.int32(-2**31)))
            hit = cnt > 0
            bsel = jnp.where(hit, j * 16 + istar, bsel)
            above_sel = jnp.where(hit, above + s_at - h_at, above_sel)
            above = jnp.where(hit, above, above + total)
            return t + 1, above, hit, bsel, above_sel

        _, _, _, bsel, above_sel = lax.while_loop(
            scan_cond, scan_body,
            (jnp.int32(0), jnp.int32(0), jnp.bool_(False), jnp.int32(0),
             jnp.int32(0)))

        prefix = jnp.where(p == 0, bsel, (prefix << 10) | bsel)
        need = need - above_sel

    return jnp.broadcast_to(prefix << 0, (16,))


def _sc_select(cent_hbm, thr_hbm, row_a, row_b, h0, h1, h2, h3, thr_v,
               sem_a, sem_b):
    cid = lax.axis_index("c")
    sid = lax.axis_index("s")
    wid = sid * 2 + cid
    row0 = wid * 2
    cp_a = pltpu.make_async_copy(cent_hbm.at[row0], row_a, sem_a)
    cp_b = pltpu.make_async_copy(cent_hbm.at[row0 + 1], row_b, sem_b)
    cp_a.start()
    cp_b.start()
    hists = [h0, h1, h2, h3]
    cp_a.wait()
    pat = _select_row(row_a, hists)
    thr_v[...] = lax.bitcast_convert_type(pat, jnp.float32)
    pltpu.sync_copy(thr_v, thr_hbm.at[row0])
    cp_b.wait()
    pat = _select_row(row_b, hists)
    thr_v[...] = lax.bitcast_convert_type(pat, jnp.float32)
    pltpu.sync_copy(thr_v, thr_hbm.at[row0 + 1])


_MESH = plsc.VectorSubcoreMesh(
    core_axis_name="c", subcore_axis_name="s", num_cores=2, num_subcores=16)

_sc_select_call = functools.partial(
    pl.kernel,
    out_type=jax.ShapeDtypeStruct((BS, 16), jnp.float32),
    mesh=_MESH,
    scratch_types=[
        pltpu.VMEM((NQ,), jnp.float32),
        pltpu.VMEM((NQ,), jnp.float32),
        pltpu.VMEM((NBUCKET,), jnp.int32),
        pltpu.VMEM((NBUCKET,), jnp.int32),
        pltpu.VMEM((NBUCKET,), jnp.int32),
        pltpu.VMEM((NBUCKET,), jnp.int32),
        pltpu.VMEM((16,), jnp.float32),
        pltpu.SemaphoreType.DMA,
        pltpu.SemaphoreType.DMA,
    ],
    compiler_params=pltpu.CompilerParams(needs_layout_passes=False),
)(_sc_select)


def kernel(bilinear_coords, boxes):
    bs, nq = bilinear_coords.shape[:2]
    x = bilinear_coords[:, :, 0]
    y = bilinear_coords[:, :, 1]
    bb = 8
    cent = pl.pallas_call(
        _cent_body,
        grid=(bs // bb,),
        in_specs=[
            pl.BlockSpec((bb, nq), lambda i: (i, 0)),
            pl.BlockSpec((bb, nq), lambda i: (i, 0)),
            pl.BlockSpec((bb, 4), lambda i: (i, 0)),
        ],
        out_specs=pl.BlockSpec((bb, nq), lambda i: (i, 0)),
        out_shape=jax.ShapeDtypeStruct((bs, nq), jnp.float32),
    )(x, y, boxes)

    thr16 = _sc_select_call(cent)

    mask = pl.pallas_call(
        _mask_body,
        grid=(bs // bb,),
        in_specs=[
            pl.BlockSpec((bb, nq), lambda i: (i, 0)),
            pl.BlockSpec((bb, 16), lambda i: (i, 0)),
        ],
        out_specs=pl.BlockSpec((bb, nq), lambda i: (i, 0)),
        out_shape=jax.ShapeDtypeStruct((bs, nq), jnp.bool_),
    )(cent, thr16)
    return cent, mask


# speedup vs baseline: 1.1052x; 1.1052x over previous
import functools

import jax
import jax.numpy as jnp
from jax import lax
from jax.experimental import pallas as pl
from jax.experimental.pallas import tpu as pltpu
from jax.experimental.pallas import tpu_sc as plsc

BS = 64
NQ = 32768
K = NQ // 16
PAD = 64
HI_BITS = 8
LO_BITS = 30 - HI_BITS
ONE_F32 = 0x3F800000


def _cent_body(x_ref, y_ref, box_ref, cent_ref):
    xb = x_ref[...]
    yb = y_ref[...]
    cx = box_ref[:, 0:1]
    cy = box_ref[:, 1:2]
    w = box_ref[:, 2:3]
    h = box_ref[:, 3:4]
    xmin = cx - w / 2.0
    ymin = cy - h / 2.0
    xmax = cx + w / 2.0
    ymax = cy + h / 2.0
    left = jnp.clip(xb - xmin, 0.0, 1.0)
    right = jnp.clip(xmax - xb, 0.0, 1.0)
    top = jnp.clip(yb - ymin, 0.0, 1.0)
    down = jnp.clip(ymax - yb, 0.0, 1.0)
    sx = (left + right) / 2.0
    dx = jnp.abs(left - right) / 2.0
    sy = (top + down) / 2.0
    dy = jnp.abs(top - down) / 2.0
    cxn = (sx - dx) / (sx + dx)
    cyn = (sy - dy) / (sy + dy)
    cent_ref[...] = jnp.sqrt(cxn * cyn)


def _mask_body(cent_ref, thr_ref, mask_ref):
    mask_ref[...] = cent_ref[...] > thr_ref[:, 0:1]


def _count_ge(buf, nblk, t):
    def body(i, acc):
        for q in range(4):
            v = buf[pl.ds((i * 4 + q) * 16, 16)]
            acc = acc + plsc.all_reduce_population_count(v >= t)
        return acc

    acc = lax.fori_loop(0, nblk, body, jnp.zeros((16,), jnp.int32))
    return jnp.max(acc)


def _compact(src, dst, nblk, keep_and_key):
    lanes = lax.iota(jnp.int32, 16)
    zeros16 = jnp.zeros((16,), jnp.int32)

    def body(i, base):
        for q in range(4):
            v = src[pl.ds((i * 4 + q) * 16, 16)]
            m, key = keep_and_key(v)
            s, _ = plsc.sort_key_val(key, key, descending=True)
            plsc.store_scatter(dst, [base + lanes], s)
            base = base + plsc.all_reduce_population_count(m)
        return base

    base = lax.fori_loop(0, nblk, body, jnp.zeros((16,), jnp.int32))
    for q in range(PAD // 16):
        plsc.store_scatter(dst, [base + lanes + q * 16], zeros16)
    return jnp.max(base)


def _select_row(row_v, cand_v):
    def keep1(u):
        m = jnp.logical_and(u > 0, u <= ONE_F32)
        return m, jnp.where(m, u, 0)

    m_cnt = _compact(row_v, cand_v, NQ // PAD, keep1)
    nblk = (m_cnt + (PAD - 1)) >> 6

    need = jnp.int32(K + 1)
    lo = jnp.int32(0)
    for bit in range(29, 29 - HI_BITS, -1):
        t = lo | (1 << bit)
        c = _count_ge(cand_v, nblk, t)
        lo = jnp.where(c >= need, t, lo)

    hi = lo + (1 << LO_BITS)
    above = _count_ge(cand_v, nblk, hi)
    need2 = need - above
    lo_eff = jnp.maximum(lo, 1)

    def keep2(u):
        m = jnp.logical_and(u >= lo_eff, u < hi)
        return m, jnp.where(m, u, 0)

    m2_cnt = _compact(cand_v, row_v, nblk, keep2)
    nblk2 = (m2_cnt + (PAD - 1)) >> 6

    res = lo
    for bit in range(LO_BITS - 1, -1, -1):
        t = res | (1 << bit)
        c = _count_ge(row_v, nblk2, t)
        res = jnp.where(c >= need2, t, res)
    return res


def _sc_select(cent_hbm, thr_hbm, row_a, row_b, cand_v, thr_v, sem_a, sem_b):
    cid = lax.axis_index("c")
    sid = lax.axis_index("s")
    wid = sid * 2 + cid
    row0 = wid * 2
    cp_a = pltpu.make_async_copy(
        cent_hbm.at[row0], row_a.at[pl.ds(0, NQ)], sem_a)
    cp_b = pltpu.make_async_copy(
        cent_hbm.at[row0 + 1], row_b.at[pl.ds(0, NQ)], sem_b)
    cp_a.start()
    cp_b.start()
    cp_a.wait()
    pat = _select_row(row_a, cand_v)
    thr_v[...] = jnp.broadcast_to(pat, (16,))
    pltpu.sync_copy(thr_v, thr_hbm.at[row0])
    cp_b.wait()
    pat = _select_row(row_b, cand_v)
    thr_v[...] = jnp.broadcast_to(pat, (16,))
    pltpu.sync_copy(thr_v, thr_hbm.at[row0 + 1])


_MESH = plsc.VectorSubcoreMesh(
    core_axis_name="c", subcore_axis_name="s", num_cores=2, num_subcores=16)

_sc_select_call = functools.partial(
    pl.kernel,
    out_type=jax.ShapeDtypeStruct((BS, 16), jnp.int32),
    mesh=_MESH,
    scratch_types=[
        pltpu.VMEM((NQ + PAD,), jnp.int32),
        pltpu.VMEM((NQ + PAD,), jnp.int32),
        pltpu.VMEM((NQ + PAD,), jnp.int32),
        pltpu.VMEM((16,), jnp.int32),
        pltpu.SemaphoreType.DMA,
        pltpu.SemaphoreType.DMA,
    ],
    compiler_params=pltpu.CompilerParams(needs_layout_passes=False),
)(_sc_select)


def kernel(bilinear_coords, boxes):
    bs, nq = bilinear_coords.shape[:2]
    x = bilinear_coords[:, :, 0]
    y = bilinear_coords[:, :, 1]
    bb = 8
    cent = pl.pallas_call(
        _cent_body,
        grid=(bs // bb,),
        in_specs=[
            pl.BlockSpec((bb, nq), lambda i: (i, 0)),
            pl.BlockSpec((bb, nq), lambda i: (i, 0)),
            pl.BlockSpec((bb, 4), lambda i: (i, 0)),
        ],
        out_specs=pl.BlockSpec((bb, nq), lambda i: (i, 0)),
        out_shape=jax.ShapeDtypeStruct((bs, nq), jnp.float32),
    )(x, y, boxes)

    thr16 = _sc_select_call(lax.bitcast_convert_type(cent, jnp.int32))
    thr = lax.bitcast_convert_type(thr16, jnp.float32)

    mask = pl.pallas_call(
        _mask_body,
        grid=(bs // bb,),
        in_specs=[
            pl.BlockSpec((bb, nq), lambda i: (i, 0)),
            pl.BlockSpec((bb, 16), lambda i: (i, 0)),
        ],
        out_specs=pl.BlockSpec((bb, nq), lambda i: (i, 0)),
        out_shape=jax.ShapeDtypeStruct((bs, nq), jnp.bool_),
    )(cent, thr)
    return cent, mask


# speedup vs baseline: 1.3377x; 1.2104x over previous
import functools

import jax
import jax.numpy as jnp
from jax import lax
from jax.experimental import pallas as pl
from jax.experimental.pallas import tpu as pltpu
from jax.experimental.pallas import tpu_sc as plsc

BS = 64
NQ = 32768
K = NQ // 16
PAD = 64
HI_BITS = 8
LO_BITS = 30 - HI_BITS
ONE_F32 = 0x3F800000


def _cent_body(x_ref, y_ref, box_ref, cent_ref):
    xb = x_ref[...]
    yb = y_ref[...]
    cx = box_ref[:, 0:1]
    cy = box_ref[:, 1:2]
    w = box_ref[:, 2:3]
    h = box_ref[:, 3:4]
    xmin = cx - w / 2.0
    ymin = cy - h / 2.0
    xmax = cx + w / 2.0
    ymax = cy + h / 2.0
    left = jnp.clip(xb - xmin, 0.0, 1.0)
    right = jnp.clip(xmax - xb, 0.0, 1.0)
    top = jnp.clip(yb - ymin, 0.0, 1.0)
    down = jnp.clip(ymax - yb, 0.0, 1.0)
    sx = (left + right) / 2.0
    dx = jnp.abs(left - right) / 2.0
    sy = (top + down) / 2.0
    dy = jnp.abs(top - down) / 2.0
    cxn = (sx - dx) / (sx + dx)
    cyn = (sy - dy) / (sy + dy)
    cent_ref[...] = jnp.sqrt(cxn * cyn)


def _mask_body(cent_ref, thr_ref, mask_ref):
    mask_ref[...] = cent_ref[...] > thr_ref[:, 0:1]


def _count_ge(buf, nblk, t):
    def body(i, acc):
        for q in range(4):
            v = buf[pl.ds((i * 4 + q) * 16, 16)]
            acc = acc + plsc.all_reduce_population_count(v >= t)
        return acc

    acc = plsc.parallel_loop(
        0, nblk, carry=jnp.zeros((16,), jnp.int32), unroll=2)(body)
    return jnp.max(acc)


def _compact(src, dst, nblk, keep_and_key):
    lanes = lax.iota(jnp.int32, 16)
    zeros16 = jnp.zeros((16,), jnp.int32)

    def body(i, base):
        for q in range(4):
            v = src[pl.ds((i * 4 + q) * 16, 16)]
            m, key = keep_and_key(v)
            mi = m.astype(jnp.int32)
            pos = base + jnp.maximum(plsc.cumsum(mi) - 1, 0)
            plsc.store_scatter(dst, [pos], key, mask=m)
            base = base + plsc.all_reduce_population_count(m)
        return base

    base = plsc.parallel_loop(
        0, nblk, carry=jnp.zeros((16,), jnp.int32), unroll=2)(body)
    for q in range(PAD // 16):
        plsc.store_scatter(dst, [base + lanes + q * 16], zeros16)
    return jnp.max(base)


def _select_row(row_v, cand_v):
    def keep1(u):
        m = jnp.logical_and(u > 0, u <= ONE_F32)
        return m, jnp.where(m, u, 0)

    m_cnt = _compact(row_v, cand_v, NQ // PAD, keep1)
    nblk = (m_cnt + (PAD - 1)) >> 6

    need = jnp.int32(K + 1)
    lo = jnp.int32(0)
    for bit in range(29, 29 - HI_BITS, -1):
        t = lo | (1 << bit)
        c = _count_ge(cand_v, nblk, t)
        lo = jnp.where(c >= need, t, lo)

    hi = lo + (1 << LO_BITS)
    above = _count_ge(cand_v, nblk, hi)
    need2 = need - above
    lo_eff = jnp.maximum(lo, 1)

    def keep2(u):
        m = jnp.logical_and(u >= lo_eff, u < hi)
        return m, jnp.where(m, u, 0)

    m2_cnt = _compact(cand_v, row_v, nblk, keep2)
    nblk2 = (m2_cnt + (PAD - 1)) >> 6

    res = lo
    for bit in range(LO_BITS - 1, -1, -1):
        t = res | (1 << bit)
        c = _count_ge(row_v, nblk2, t)
        res = jnp.where(c >= need2, t, res)
    return res


def _sc_select(cent_hbm, thr_hbm, row_a, row_b, cand_v, thr_v, sem_a, sem_b):
    cid = lax.axis_index("c")
    sid = lax.axis_index("s")
    wid = sid * 2 + cid
    row0 = wid * 2
    cp_a = pltpu.make_async_copy(
        cent_hbm.at[row0], row_a.at[pl.ds(0, NQ)], sem_a)
    cp_b = pltpu.make_async_copy(
        cent_hbm.at[row0 + 1], row_b.at[pl.ds(0, NQ)], sem_b)
    cp_a.start()
    cp_b.start()
    cp_a.wait()
    pat = _select_row(row_a, cand_v)
    thr_v[...] = jnp.broadcast_to(pat, (16,))
    pltpu.sync_copy(thr_v, thr_hbm.at[row0])
    cp_b.wait()
    pat = _select_row(row_b, cand_v)
    thr_v[...] = jnp.broadcast_to(pat, (16,))
    pltpu.sync_copy(thr_v, thr_hbm.at[row0 + 1])


_MESH = plsc.VectorSubcoreMesh(
    core_axis_name="c", subcore_axis_name="s", num_cores=2, num_subcores=16)

_sc_select_call = functools.partial(
    pl.kernel,
    out_type=jax.ShapeDtypeStruct((BS, 16), jnp.int32),
    mesh=_MESH,
    scratch_types=[
        pltpu.VMEM((NQ + PAD,), jnp.int32),
        pltpu.VMEM((NQ + PAD,), jnp.int32),
        pltpu.VMEM((NQ + PAD,), jnp.int32),
        pltpu.VMEM((16,), jnp.int32),
        pltpu.SemaphoreType.DMA,
        pltpu.SemaphoreType.DMA,
    ],
    compiler_params=pltpu.CompilerParams(needs_layout_passes=False),
)(_sc_select)


def kernel(bilinear_coords, boxes):
    bs, nq = bilinear_coords.shape[:2]
    x = bilinear_coords[:, :, 0]
    y = bilinear_coords[:, :, 1]
    bb = 8
    cent = pl.pallas_call(
        _cent_body,
        grid=(bs // bb,),
        in_specs=[
            pl.BlockSpec((bb, nq), lambda i: (i, 0)),
            pl.BlockSpec((bb, nq), lambda i: (i, 0)),
            pl.BlockSpec((bb, 4), lambda i: (i, 0)),
        ],
        out_specs=pl.BlockSpec((bb, nq), lambda i: (i, 0)),
        out_shape=jax.ShapeDtypeStruct((bs, nq), jnp.float32),
    )(x, y, boxes)

    thr16 = _sc_select_call(lax.bitcast_convert_type(cent, jnp.int32))
    thr = lax.bitcast_convert_type(thr16, jnp.float32)

    mask = pl.pallas_call(
        _mask_body,
        grid=(bs // bb,),
        in_specs=[
            pl.BlockSpec((bb, nq), lambda i: (i, 0)),
            pl.BlockSpec((bb, 16), lambda i: (i, 0)),
        ],
        out_specs=pl.BlockSpec((bb, nq), lambda i: (i, 0)),
        out_shape=jax.ShapeDtypeStruct((bs, nq), jnp.bool_),
    )(cent, thr)
    return cent, mask
